# Initial kernel scaffold; baseline (speedup 1.0000x reference)
import jax
import jax.numpy as jnp
from jax.experimental import pallas as pl


def kernel(state, edge_index, action, W_gcn, b_gcn, W1, b1, W2, b2, W3, b3):
    raise NotImplementedError("write your pallas kernel here")


# speedup vs baseline: 11.0945x; 11.0945x over previous
import functools

import jax
import jax.numpy as jnp
from jax import lax
from jax.experimental import pallas as pl
from jax.experimental.pallas import tpu as pltpu
from jax.experimental.pallas import tpu_sc as plsc

N = 10000
D = 128
E = 320000
H = 256
ACT = 8

NC = 2
NS = 16
NW = NC * NS
CH = 128
EP = 10240
EPAD = NW * EP
NCH = EP // CH
NPAD = 10240
RPT = NPAD // NS
DUMMY = N + 100

def _deg_body(dst_hbm, out_hbm, dstbuf, locdeg):
    c = lax.axis_index("c")
    s = lax.axis_index("s")
    wid = s * NC + c
    zero16 = jnp.zeros((16,), jnp.float32)
    ones16 = jnp.ones((16,), jnp.float32)

    def zb(i, carry):
        locdeg[pl.ds(i * 16, 16)] = zero16
        return carry

    lax.fori_loop(0, NPAD // 16, zb, 0)
    pltpu.sync_copy(dst_hbm.at[pl.ds(wid * EP, EP)], dstbuf)

    def ab(i, carry):
        idx = dstbuf[pl.ds(i * 16, 16)]
        plsc.addupdate_scatter(locdeg, [idx], ones16)
        return carry

    lax.fori_loop(0, EP // 16, ab, 0)
    pltpu.sync_copy(locdeg, out_hbm.at[c].at[s])


def _msg_body(src_hbm, dst_hbm, z_hbm, out_hbm, sidx, didx, rows, acc,
              sem0, sem1):
    c = lax.axis_index("c")
    s = lax.axis_index("s")
    wid = s * NC + c
    base = wid * EP
    zero16 = jnp.zeros((16,), jnp.float32)
    sems = (sem0, sem1)

    def zb(i, carry):
        r = i // (D // 16)
        k = i % (D // 16)
        rows[0, r, pl.ds(k * 16, 16)] = zero16
        return carry

    lax.fori_loop(0, CH * D // 16, zb, 0)
    for q in range(RPT // CH):
        pltpu.sync_copy(rows.at[0], acc.at[pl.ds(s * RPT + q * CH, CH)])
    plsc.subcore_barrier()

    for b in range(2):
        pltpu.sync_copy(src_hbm.at[pl.ds(base + b * CH, CH)], sidx.at[b])
        pltpu.sync_copy(dst_hbm.at[pl.ds(base + b * CH, CH)], didx.at[b])
        pltpu.make_async_copy(z_hbm.at[sidx.at[b]], rows.at[b], sems[b]).start()

    def mb(g, carry):
        for b in range(2):
            j = g * 2 + b
            pltpu.make_async_copy(z_hbm.at[sidx.at[b]], rows.at[b],
                                  sems[b]).wait()
            pltpu.sync_copy(rows.at[b], acc.at[didx.at[b]], add=True)
            nj = j + 2

            @pl.when(nj < NCH)
            def _prefetch():
                pltpu.sync_copy(src_hbm.at[pl.ds(base + nj * CH, CH)],
                                sidx.at[b])
                pltpu.sync_copy(dst_hbm.at[pl.ds(base + nj * CH, CH)],
                                didx.at[b])
                pltpu.make_async_copy(z_hbm.at[sidx.at[b]], rows.at[b],
                                      sems[b]).start()
        return carry

    lax.fori_loop(0, NCH // 2, mb, 0)
    plsc.subcore_barrier()
    pltpu.sync_copy(acc.at[pl.ds(s * RPT, RPT)],
                    out_hbm.at[c].at[pl.ds(s * RPT, RPT)])


@functools.cache
def _sc_kernels():
    mesh = plsc.VectorSubcoreMesh(core_axis_name="c", subcore_axis_name="s")
    cparams = pltpu.CompilerParams(needs_layout_passes=False)
    deg_kernel = pl.kernel(
        _deg_body,
        out_type=jax.ShapeDtypeStruct((NC, NS, NPAD), jnp.float32),
        mesh=mesh,
        scratch_types=[
            pltpu.VMEM((EP,), jnp.int32),
            pltpu.VMEM((NPAD,), jnp.float32),
        ],
        compiler_params=cparams,
    )
    msg_kernel = pl.kernel(
        _msg_body,
        out_type=jax.ShapeDtypeStruct((NC, NPAD, D), jnp.float32),
        mesh=mesh,
        scratch_types=[
            pltpu.VMEM((2, CH), jnp.int32),
            pltpu.VMEM((2, CH), jnp.int32),
            pltpu.VMEM((2, CH, D), jnp.float32),
            pltpu.VMEM_SHARED((NPAD, D), jnp.float32),
            pltpu.SemaphoreType.DMA,
            pltpu.SemaphoreType.DMA,
        ],
        compiler_params=cparams,
    )
    return deg_kernel, msg_kernel


def _dinv_body(dp_ref, dinv_ref):
    tot = jnp.sum(dp_ref[...], axis=0, keepdims=True) + 1.0
    dinv_ref[...] = lax.rsqrt(tot)


def _dinv_call(dp):
    return pl.pallas_call(
        _dinv_body,
        out_shape=jax.ShapeDtypeStruct((1, NPAD), jnp.float32),
    )(dp)


_BM = 2000


def _z_body(state_ref, w_ref, dinv_ref, z_ref):
    z_ref[...] = jnp.dot(state_ref[...], w_ref[...],
                         preferred_element_type=jnp.float32, precision=lax.Precision.HIGHEST) * dinv_ref[...]


def _z_call(state, w_gcn, dinv_col):
    grid = (N // _BM,)
    return pl.pallas_call(
        _z_body,
        grid=grid,
        in_specs=[
            pl.BlockSpec((_BM, D), lambda i: (i, 0)),
            pl.BlockSpec((D, D), lambda i: (0, 0)),
            pl.BlockSpec((_BM, 1), lambda i: (i, 0)),
        ],
        out_specs=pl.BlockSpec((_BM, D), lambda i: (i, 0)),
        out_shape=jax.ShapeDtypeStruct((N, D), jnp.float32),
    )(state, w_gcn, dinv_col)


_BMH = 2048
_GH = _BMH // ACT


def _head_body(acc0, acc1, z, state, dinv, aflat, bg, w1, b1, w2, b2, w3, b3,
               out):
    x = (acc0[...] + acc1[...] + z[...]) * dinv[...] + bg[...]
    x = jnp.maximum(x, 0.0) + state[...]
    gi = lax.broadcasted_iota(jnp.int32, (_GH, _BMH), 0)
    ri = lax.broadcasted_iota(jnp.int32, (_GH, _BMH), 1)
    sel = (ri // ACT) == gi
    smat = jnp.where(sel, aflat[...] * 10.0, 0.0)
    y = jnp.dot(smat, x, preferred_element_type=jnp.float32, precision=lax.Precision.HIGHEST)
    h = jnp.maximum(
        jnp.dot(y, w1[...], preferred_element_type=jnp.float32, precision=lax.Precision.HIGHEST) + b1[...], 0.0)
    h = jnp.maximum(
        jnp.dot(h, w2[...], preferred_element_type=jnp.float32, precision=lax.Precision.HIGHEST) + b2[...], 0.0)
    out[...] = jnp.dot(h, w3[...], preferred_element_type=jnp.float32, precision=lax.Precision.HIGHEST) + b3[...]


def _head_call(acc0, acc1, z, state, dinv_col, aflat, b_gcn, w1, b1, w2, b2,
               w3, b3):
    grid = (NPAD // _BMH,)
    row_spec = pl.BlockSpec((_BMH, D), lambda i: (i, 0))
    return pl.pallas_call(
        _head_body,
        grid=grid,
        in_specs=[
            row_spec, row_spec, row_spec, row_spec,
            pl.BlockSpec((_BMH, 1), lambda i: (i, 0)),
            pl.BlockSpec((1, _BMH), lambda i: (0, i)),
            pl.BlockSpec((1, D), lambda i: (0, 0)),
            pl.BlockSpec((D, H), lambda i: (0, 0)),
            pl.BlockSpec((1, H), lambda i: (0, 0)),
            pl.BlockSpec((H, H), lambda i: (0, 0)),
            pl.BlockSpec((1, H), lambda i: (0, 0)),
            pl.BlockSpec((H, 1), lambda i: (0, 0)),
            pl.BlockSpec((1, 1), lambda i: (0, 0)),
        ],
        out_specs=pl.BlockSpec((_GH, 1), lambda i: (i, 0)),
        out_shape=jax.ShapeDtypeStruct((NPAD // ACT, 1), jnp.float32),
    )(acc0, acc1, z, state, dinv_col, aflat, b_gcn, w1, b1, w2, b2, w3, b3)


def kernel(state, edge_index, action, W_gcn, b_gcn, W1, b1, W2, b2, W3, b3):
    src = edge_index[0]
    dst = edge_index[1]
    pad = EPAD - E
    src_p = jnp.concatenate([src, jnp.zeros((pad,), jnp.int32)])
    dst_p = jnp.concatenate([dst, jnp.full((pad,), DUMMY, jnp.int32)])

    deg_kernel, msg_kernel = _sc_kernels()
    deg_parts = deg_kernel(dst_p)
    dinv_row = _dinv_call(deg_parts.reshape(NW, NPAD))
    dinv_full = dinv_row.reshape(NPAD, 1)
    dinv_col = dinv_full[:N]

    z = _z_call(state, W_gcn, dinv_col)
    accp = msg_kernel(src_p, dst_p, z)

    rpad = NPAD - N
    zrows = jnp.zeros((rpad, D), jnp.float32)
    z_p = jnp.concatenate([z, zrows])
    state_p = jnp.concatenate([state, zrows])
    aflat = jnp.concatenate(
        [action.reshape(1, N), jnp.zeros((1, rpad), jnp.float32)], axis=1)

    out = _head_call(
        accp[0], accp[1], z_p, state_p, dinv_full, aflat,
        b_gcn.reshape(1, D),
        W1, b1.reshape(1, H), W2, b2.reshape(1, H), W3, b3.reshape(1, 1))
    return out.reshape(NPAD // ACT)[:N // ACT]


# speedup vs baseline: 13.2007x; 1.1898x over previous
import functools

import jax
import jax.numpy as jnp
from jax import lax
from jax.experimental import pallas as pl
from jax.experimental.pallas import tpu as pltpu
from jax.experimental.pallas import tpu_sc as plsc

N = 10000
D = 128
E = 320000
H = 256
ACT = 8

NC = 2
NS = 16
NW = NC * NS
CH = 128
EP = 10240
EPAD = NW * EP
NCH = EP // CH
NPAD = 10240
RPT = NPAD // NS
DUMMY = N + 100

def _deg_body(dst_hbm, out_hbm, dstbuf, locdeg):
    c = lax.axis_index("c")
    s = lax.axis_index("s")
    wid = s * NC + c
    zero16 = jnp.zeros((16,), jnp.float32)
    ones16 = jnp.ones((16,), jnp.float32)

    def zb(i, carry):
        locdeg[pl.ds(i * 16, 16)] = zero16
        return carry

    lax.fori_loop(0, NPAD // 16, zb, 0)
    pltpu.sync_copy(dst_hbm.at[pl.ds(wid * EP, EP)], dstbuf)

    def ab(i, carry):
        idx = dstbuf[pl.ds(i * 16, 16)]
        plsc.addupdate_scatter(locdeg, [idx], ones16)
        return carry

    lax.fori_loop(0, EP // 16, ab, 0)
    pltpu.sync_copy(locdeg, out_hbm.at[c].at[s])


def _msg_body(src_hbm, dst_hbm, z_hbm, out_hbm, sidx, didx, rows, acc,
              sem0, sem1):
    c = lax.axis_index("c")
    s = lax.axis_index("s")
    wid = s * NC + c
    base = wid * EP
    zero16 = jnp.zeros((16,), jnp.float32)
    sems = (sem0, sem1)

    def zb(i, carry):
        r = i // (D // 16)
        k = i % (D // 16)
        rows[0, r, pl.ds(k * 16, 16)] = zero16
        return carry

    lax.fori_loop(0, CH * D // 16, zb, 0)
    for q in range(RPT // CH):
        pltpu.sync_copy(rows.at[0], acc.at[pl.ds(s * RPT + q * CH, CH)])
    plsc.subcore_barrier()

    for b in range(2):
        pltpu.sync_copy(src_hbm.at[pl.ds(base + b * CH, CH)], sidx.at[b])
        pltpu.sync_copy(dst_hbm.at[pl.ds(base + b * CH, CH)], didx.at[b])
        pltpu.make_async_copy(z_hbm.at[sidx.at[b]], rows.at[b], sems[b]).start()

    def mb(g, carry):
        for b in range(2):
            j = g * 2 + b
            pltpu.make_async_copy(z_hbm.at[sidx.at[b]], rows.at[b],
                                  sems[b]).wait()
            pltpu.sync_copy(rows.at[b], acc.at[didx.at[b]], add=True)
            nj = j + 2

            @pl.when(nj < NCH)
            def _prefetch():
                pltpu.sync_copy(src_hbm.at[pl.ds(base + nj * CH, CH)],
                                sidx.at[b])
                pltpu.sync_copy(dst_hbm.at[pl.ds(base + nj * CH, CH)],
                                didx.at[b])
                pltpu.make_async_copy(z_hbm.at[sidx.at[b]], rows.at[b],
                                      sems[b]).start()
        return carry

    lax.fori_loop(0, NCH // 2, mb, 0)
    plsc.subcore_barrier()
    pltpu.sync_copy(acc.at[pl.ds(s * RPT, RPT)],
                    out_hbm.at[c].at[pl.ds(s * RPT, RPT)])


@functools.cache
def _sc_kernels():
    mesh = plsc.VectorSubcoreMesh(core_axis_name="c", subcore_axis_name="s")
    cparams = pltpu.CompilerParams(needs_layout_passes=False)
    deg_kernel = pl.kernel(
        _deg_body,
        out_type=jax.ShapeDtypeStruct((NC, NS, NPAD), jnp.float32),
        mesh=mesh,
        scratch_types=[
            pltpu.VMEM((EP,), jnp.int32),
            pltpu.VMEM((NPAD,), jnp.float32),
        ],
        compiler_params=cparams,
    )
    msg_kernel = pl.kernel(
        _msg_body,
        out_type=jax.ShapeDtypeStruct((NC, NPAD, D), jnp.float32),
        mesh=mesh,
        scratch_types=[
            pltpu.VMEM((2, CH), jnp.int32),
            pltpu.VMEM((2, CH), jnp.int32),
            pltpu.VMEM((2, CH, D), jnp.float32),
            pltpu.VMEM_SHARED((NPAD, D), jnp.float32),
            pltpu.SemaphoreType.DMA,
            pltpu.SemaphoreType.DMA,
        ],
        compiler_params=cparams,
    )
    return deg_kernel, msg_kernel


def _dinv_body(dp_ref, dinv_ref):
    tot = jnp.sum(dp_ref[...], axis=0, keepdims=True) + 1.0
    dinv_ref[...] = lax.rsqrt(tot)


def _dinv_call(dp):
    return pl.pallas_call(
        _dinv_body,
        out_shape=jax.ShapeDtypeStruct((1, NPAD), jnp.float32),
    )(dp)


_BM = 2000


def _z_body(state_ref, w_ref, dinv_ref, z_ref):
    z_ref[...] = jnp.dot(state_ref[...], w_ref[...],
                         preferred_element_type=jnp.float32, precision=lax.Precision.HIGHEST) * dinv_ref[...]


def _z_call(state, w_gcn, dinv_col):
    grid = (N // _BM,)
    return pl.pallas_call(
        _z_body,
        grid=grid,
        in_specs=[
            pl.BlockSpec((_BM, D), lambda i: (i, 0)),
            pl.BlockSpec((D, D), lambda i: (0, 0)),
            pl.BlockSpec((_BM, 1), lambda i: (i, 0)),
        ],
        out_specs=pl.BlockSpec((_BM, D), lambda i: (i, 0)),
        out_shape=jax.ShapeDtypeStruct((N, D), jnp.float32),
    )(state, w_gcn, dinv_col)


_BMH = 2048
_GH = _BMH // ACT


def _head_body(acc0, acc1, z, state, dinv, aflat, bg, w1, b1, w2, b2, w3, b3,
               out):
    x = (acc0[...] + acc1[...] + z[...]) * dinv[...] + bg[...]
    x = jnp.maximum(x, 0.0) + state[...]
    gi = lax.broadcasted_iota(jnp.int32, (_GH, _BMH), 0)
    ri = lax.broadcasted_iota(jnp.int32, (_GH, _BMH), 1)
    sel = (ri // ACT) == gi
    smat = jnp.where(sel, aflat[...] * 10.0, 0.0)
    y = jnp.dot(smat, x, preferred_element_type=jnp.float32, precision=lax.Precision.HIGHEST)
    h = jnp.maximum(
        jnp.dot(y, w1[...], preferred_element_type=jnp.float32, precision=lax.Precision.HIGHEST) + b1[...], 0.0)
    h = jnp.maximum(
        jnp.dot(h, w2[...], preferred_element_type=jnp.float32, precision=lax.Precision.HIGHEST) + b2[...], 0.0)
    out[...] = jnp.dot(h, w3[...], preferred_element_type=jnp.float32, precision=lax.Precision.HIGHEST) + b3[...]


def _head_call(acc0, acc1, z, state, dinv_col, aflat, b_gcn, w1, b1, w2, b2,
               w3, b3):
    grid = (NPAD // _BMH,)
    row_spec = pl.BlockSpec((_BMH, D), lambda i: (i, 0))
    return pl.pallas_call(
        _head_body,
        grid=grid,
        in_specs=[
            row_spec, row_spec, row_spec, row_spec,
            pl.BlockSpec((_BMH, 1), lambda i: (i, 0)),
            pl.BlockSpec((1, _BMH), lambda i: (0, i)),
            pl.BlockSpec((1, D), lambda i: (0, 0)),
            pl.BlockSpec((D, H), lambda i: (0, 0)),
            pl.BlockSpec((1, H), lambda i: (0, 0)),
            pl.BlockSpec((H, H), lambda i: (0, 0)),
            pl.BlockSpec((1, H), lambda i: (0, 0)),
            pl.BlockSpec((H, 1), lambda i: (0, 0)),
            pl.BlockSpec((1, 1), lambda i: (0, 0)),
        ],
        out_specs=pl.BlockSpec((_GH, 1), lambda i: (i, 0)),
        out_shape=jax.ShapeDtypeStruct((NPAD // ACT, 1), jnp.float32),
    )(acc0, acc1, z, state, dinv_col, aflat, b_gcn, w1, b1, w2, b2, w3, b3)


def kernel(state, edge_index, action, W_gcn, b_gcn, W1, b1, W2, b2, W3, b3):
    src = edge_index[0]
    dst = edge_index[1]
    wpad = EP - E // NW
    src_p = jnp.concatenate(
        [src.reshape(NW, E // NW),
         jnp.zeros((NW, wpad), jnp.int32)], axis=1).reshape(-1)
    dummy_rows = jnp.broadcast_to(
        jnp.arange(N, N + wpad, dtype=jnp.int32), (NW, wpad))
    dst_p = jnp.concatenate(
        [dst.reshape(NW, E // NW), dummy_rows], axis=1).reshape(-1)

    deg_kernel, msg_kernel = _sc_kernels()
    deg_parts = deg_kernel(dst_p)
    dinv_row = _dinv_call(deg_parts.reshape(NW, NPAD))
    dinv_full = dinv_row.reshape(NPAD, 1)
    dinv_col = dinv_full[:N]

    z = _z_call(state, W_gcn, dinv_col)
    accp = msg_kernel(src_p, dst_p, z)

    rpad = NPAD - N
    zrows = jnp.zeros((rpad, D), jnp.float32)
    z_p = jnp.concatenate([z, zrows])
    state_p = jnp.concatenate([state, zrows])
    aflat = jnp.concatenate(
        [action.reshape(1, N), jnp.zeros((1, rpad), jnp.float32)], axis=1)

    out = _head_call(
        accp[0], accp[1], z_p, state_p, dinv_full, aflat,
        b_gcn.reshape(1, D),
        W1, b1.reshape(1, H), W2, b2.reshape(1, H), W3, b3.reshape(1, 1))
    return out.reshape(NPAD // ACT)[:N // ACT]
